# speedup vs baseline: 3.0595x; 3.0595x over previous
import functools

import jax
import jax.numpy as jnp
from jax import lax
from jax.experimental import pallas as pl
from jax.experimental.pallas import tpu as pltpu
from jax.experimental.pallas import tpu_sc as plsc

_NC = 2
_NS = 16
_NW = _NC * _NS
_L = 16


def _make_kernel(n_nodes, d_feat, batch, s):
    chunk = 4
    rows = chunk * s
    assert rows <= 128
    assert batch % chunk == 0
    nchunks = batch // chunk
    ncols = d_feat // _L
    inv_s = 1.0 / float(s)

    mesh = plsc.VectorSubcoreMesh(core_axis_name="c", subcore_axis_name="s")

    @functools.partial(
        pl.kernel,
        out_type=jax.ShapeDtypeStruct((batch, d_feat), jnp.float32),
        mesh=mesh,
        scratch_types=[
            pltpu.VMEM((rows,), jnp.int32),
            pltpu.VMEM((rows, d_feat), jnp.float32),
            pltpu.VMEM((chunk, d_feat), jnp.float32),
            pltpu.SemaphoreType.DMA,
        ],
    )
    def k(feat_hbm, tn_hbm, out_hbm, idx_v, rows_v, acc_v, sem):
        wid = lax.axis_index("s") * _NC + lax.axis_index("c")
        base_cnt = nchunks // _NW
        extra = nchunks % _NW
        cnt = base_cnt + jnp.where(wid < extra, 1, 0)
        start = wid * base_cnt + jnp.minimum(wid, extra)

        def chunk_body(i, carry):
            c = start + i
            pltpu.sync_copy(tn_hbm.at[c], idx_v)
            pltpu.async_copy(feat_hbm.at[idx_v], rows_v, sem).wait()
            for r in range(chunk):
                def s_body(j, acc):
                    row = r * s + j
                    return tuple(
                        acc[col] + rows_v[row, pl.ds(col * _L, _L)]
                        for col in range(ncols)
                    )
                sums = lax.fori_loop(
                    0, s, s_body,
                    tuple(jnp.zeros((_L,), jnp.float32) for _ in range(ncols)),
                )
                for col in range(ncols):
                    acc_v[r, pl.ds(col * _L, _L)] = sums[col] * inv_s
            pltpu.sync_copy(acc_v, out_hbm.at[pl.ds(c * chunk, chunk)])
            return carry

        lax.fori_loop(0, cnt, chunk_body, 0)

    return k


def kernel(features, nodes, to_neighs, num_sample):
    del nodes, num_sample
    batch, s = to_neighs.shape
    n_nodes, d_feat = features.shape
    k = _make_kernel(n_nodes, d_feat, batch, s)
    chunk = 4
    tn = to_neighs.astype(jnp.int32).reshape(batch // chunk, chunk * s)
    return k(features, tn)


# speedup vs baseline: 6.1248x; 2.0019x over previous
import functools

import jax
import jax.numpy as jnp
from jax import lax
from jax.experimental import pallas as pl
from jax.experimental.pallas import tpu as pltpu
from jax.experimental.pallas import tpu_sc as plsc

_NC = 2
_NS = 16
_NW = _NC * _NS
_L = 16


def _make_kernel(n_nodes, d_feat, batch, s):
    chunk = 4
    rows = chunk * s
    assert rows <= 128
    assert batch % chunk == 0
    nchunks = batch // chunk
    nmax = nchunks // _NW + (1 if nchunks % _NW else 0)
    ncols = d_feat // _L
    inv_s = 1.0 / float(s)

    mesh = plsc.VectorSubcoreMesh(core_axis_name="c", subcore_axis_name="s")

    @functools.partial(
        pl.kernel,
        out_type=jax.ShapeDtypeStruct((batch, d_feat), jnp.float32),
        mesh=mesh,
        scratch_types=[
            pltpu.VMEM((nmax * rows,), jnp.int32),
            pltpu.VMEM((rows, d_feat), jnp.float32),
            pltpu.VMEM((rows, d_feat), jnp.float32),
            pltpu.VMEM((chunk, d_feat), jnp.float32),
            pltpu.SemaphoreType.DMA,
            pltpu.SemaphoreType.DMA,
        ],
    )
    def k(feat_hbm, tn_hbm, out_hbm, idx_all, rows0_v, rows1_v, acc_v,
          sem0, sem1):
        wid = lax.axis_index("s") * _NC + lax.axis_index("c")
        base_cnt = nchunks // _NW
        extra = nchunks % _NW
        cnt = base_cnt + jnp.where(wid < extra, 1, 0)
        start = wid * base_cnt + jnp.minimum(wid, extra)
        ibase = jnp.minimum(start, nchunks - nmax)
        off = start - ibase
        pltpu.sync_copy(tn_hbm.at[pl.ds(ibase * rows, nmax * rows)], idx_all)

        def issue(c, rows_v, sem):
            pltpu.async_copy(
                feat_hbm.at[idx_all.at[pl.ds((off + c) * rows, rows)]],
                rows_v, sem)

        def wait(rows_v, sem):
            pltpu.make_async_copy(
                feat_hbm.at[idx_all.at[pl.ds(0, rows)]], rows_v, sem).wait()

        def compute(c, rows_v):
            for r in range(chunk):
                def s_body(j, acc):
                    row = r * s + j
                    return tuple(
                        acc[col] + rows_v[row, pl.ds(col * _L, _L)]
                        for col in range(ncols)
                    )
                sums = lax.fori_loop(
                    0, s, s_body,
                    tuple(jnp.zeros((_L,), jnp.float32) for _ in range(ncols)),
                )
                for col in range(ncols):
                    acc_v[r, pl.ds(col * _L, _L)] = sums[col] * inv_s
            pltpu.sync_copy(acc_v,
                            out_hbm.at[pl.ds((start + c) * chunk, chunk)])

        issue(0, rows0_v, sem0)
        npairs = cnt // 2

        def pair_body(p, carry):
            c0 = 2 * p
            issue(c0 + 1, rows1_v, sem1)
            wait(rows0_v, sem0)
            compute(c0, rows0_v)

            @pl.when(c0 + 2 < cnt)
            def _():
                issue(c0 + 2, rows0_v, sem0)

            wait(rows1_v, sem1)
            compute(c0 + 1, rows1_v)
            return carry

        lax.fori_loop(0, npairs, pair_body, 0)

        @pl.when(cnt % 2 == 1)
        def _():
            wait(rows0_v, sem0)
            compute(cnt - 1, rows0_v)

    return k


def kernel(features, nodes, to_neighs, num_sample):
    del nodes, num_sample
    batch, s = to_neighs.shape
    n_nodes, d_feat = features.shape
    k = _make_kernel(n_nodes, d_feat, batch, s)
    tn = to_neighs.astype(jnp.int32).reshape(batch * s)
    return k(features, tn)


# speedup vs baseline: 8.0650x; 1.3168x over previous
import functools

import jax
import jax.numpy as jnp
from jax import lax
from jax.experimental import pallas as pl
from jax.experimental.pallas import tpu as pltpu
from jax.experimental.pallas import tpu_sc as plsc

_NC = 2
_NS = 16
_NW = _NC * _NS
_L = 16


def _make_kernel(n_nodes, d_feat, batch, s):
    chunk = 4
    rows = chunk * s
    assert rows <= 128
    assert batch % chunk == 0
    nchunks = batch // chunk
    nmax = nchunks // _NW + (1 if nchunks % _NW else 0)
    ncols = d_feat // _L
    inv_s = 1.0 / float(s)

    mesh = plsc.VectorSubcoreMesh(core_axis_name="c", subcore_axis_name="s")

    @functools.partial(
        pl.kernel,
        out_type=jax.ShapeDtypeStruct((batch, d_feat), jnp.float32),
        mesh=mesh,
        scratch_types=[
            pltpu.VMEM((nmax * rows,), jnp.int32),
            pltpu.VMEM((rows, d_feat), jnp.float32),
            pltpu.VMEM((rows, d_feat), jnp.float32),
            pltpu.VMEM((rows, d_feat), jnp.float32),
            pltpu.VMEM((rows, d_feat), jnp.float32),
            pltpu.VMEM((chunk, d_feat), jnp.float32),
            pltpu.SemaphoreType.DMA,
            pltpu.SemaphoreType.DMA,
            pltpu.SemaphoreType.DMA,
            pltpu.SemaphoreType.DMA,
        ],
    )
    def k(feat_hbm, tn_hbm, out_hbm, idx_all, rows0_v, rows1_v, rows2_v,
          rows3_v, acc_v, sem0, sem1, sem2, sem3):
        bufs = [rows0_v, rows1_v, rows2_v, rows3_v]
        sems = [sem0, sem1, sem2, sem3]
        nbuf = 4
        wid = lax.axis_index("s") * _NC + lax.axis_index("c")
        base_cnt = nchunks // _NW
        extra = nchunks % _NW
        cnt = base_cnt + jnp.where(wid < extra, 1, 0)
        start = wid * base_cnt + jnp.minimum(wid, extra)
        ibase = jnp.minimum(start, nchunks - nmax)
        off = start - ibase
        pltpu.sync_copy(tn_hbm.at[pl.ds(ibase * rows, nmax * rows)], idx_all)

        def issue(c, rows_v, sem):
            pltpu.async_copy(
                feat_hbm.at[idx_all.at[pl.ds((off + c) * rows, rows)]],
                rows_v, sem)

        def wait(rows_v, sem):
            pltpu.make_async_copy(
                feat_hbm.at[idx_all.at[pl.ds(0, rows)]], rows_v, sem).wait()

        def compute(c, rows_v):
            for r in range(chunk):
                def s_body(j, acc):
                    row = r * s + j
                    return tuple(
                        acc[col] + rows_v[row, pl.ds(col * _L, _L)]
                        for col in range(ncols)
                    )
                sums = lax.fori_loop(
                    0, s, s_body,
                    tuple(jnp.zeros((_L,), jnp.float32) for _ in range(ncols)),
                )
                for col in range(ncols):
                    acc_v[r, pl.ds(col * _L, _L)] = sums[col] * inv_s
            pltpu.sync_copy(acc_v,
                            out_hbm.at[pl.ds((start + c) * chunk, chunk)])

        for u in range(nbuf - 1):
            issue(u, bufs[u], sems[u])

        nq = cnt // nbuf

        def quad_body(q, carry):
            c0 = nbuf * q
            for u in range(nbuf):
                pb = (u + nbuf - 1) % nbuf

                @pl.when(c0 + u + nbuf - 1 < cnt)
                def _():
                    issue(c0 + u + nbuf - 1, bufs[pb], sems[pb])

                wait(bufs[u], sems[u])
                compute(c0 + u, bufs[u])
            return carry

        lax.fori_loop(0, nq, quad_body, 0)

        rem_base = nq * nbuf
        for u in range(nbuf - 1):
            @pl.when(rem_base + u < cnt)
            def _():
                wait(bufs[u], sems[u])
                compute(rem_base + u, bufs[u])

    return k


def kernel(features, nodes, to_neighs, num_sample):
    del nodes, num_sample
    batch, s = to_neighs.shape
    n_nodes, d_feat = features.shape
    k = _make_kernel(n_nodes, d_feat, batch, s)
    tn = to_neighs.astype(jnp.int32).reshape(batch * s)
    return k(features, tn)
